# Initial kernel scaffold; baseline (speedup 1.0000x reference)
import jax
import jax.numpy as jnp
from jax.experimental import pallas as pl


def kernel(frames):
    raise NotImplementedError("write your pallas kernel here")


# speedup vs baseline: 2.8719x; 2.8719x over previous
import jax
import jax.numpy as jnp
from jax.experimental import pallas as pl


def _copy2_body(x_ref, slow_ref, fast_ref):
    v = x_ref[...]
    slow_ref[...] = v
    fast_ref[...] = v


def kernel(frames):
    C, T, H, W = frames.shape
    TB = 8
    grid = (C, T // TB)
    blk = (1, TB, H, W)
    idx_map = lambda c, t: (c, t, 0, 0)
    slow, fast = pl.pallas_call(
        _copy2_body,
        grid=grid,
        in_specs=[pl.BlockSpec(blk, idx_map)],
        out_specs=[pl.BlockSpec(blk, idx_map), pl.BlockSpec(blk, idx_map)],
        out_shape=[
            jax.ShapeDtypeStruct(frames.shape, frames.dtype),
            jax.ShapeDtypeStruct(frames.shape, frames.dtype),
        ],
    )(frames)
    return (slow, fast)


# speedup vs baseline: 2.9599x; 1.0306x over previous
import jax
import jax.numpy as jnp
from jax.experimental import pallas as pl


def _copy2_body(x_ref, slow_ref, fast_ref):
    v = x_ref[...]
    slow_ref[...] = v
    fast_ref[...] = v


def kernel(frames):
    C, T, H, W = frames.shape
    TB = 16
    grid = (C, T // TB)
    blk = (1, TB, H, W)
    idx_map = lambda c, t: (c, t, 0, 0)
    slow, fast = pl.pallas_call(
        _copy2_body,
        grid=grid,
        in_specs=[pl.BlockSpec(blk, idx_map)],
        out_specs=[pl.BlockSpec(blk, idx_map), pl.BlockSpec(blk, idx_map)],
        out_shape=[
            jax.ShapeDtypeStruct(frames.shape, frames.dtype),
            jax.ShapeDtypeStruct(frames.shape, frames.dtype),
        ],
    )(frames)
    return (slow, fast)
